# Initial kernel scaffold; baseline (speedup 1.0000x reference)
import jax
import jax.numpy as jnp
from jax.experimental import pallas as pl


def kernel(x, pos, batch, W1, b1, W2, b2):
    raise NotImplementedError("write your pallas kernel here")


# speedup vs baseline: 1.0032x; 1.0032x over previous
import functools

import jax
import jax.numpy as jnp
from jax.experimental import pallas as pl

RATIO = 0.25
R = 0.2
MAX_NBRS = 128
N = 10000
D_FEAT = 128

QB = 32


def _fps(pos, n_samples):
    idx0 = jnp.zeros((n_samples,), dtype=jnp.int32)
    d0 = jnp.sum((pos - pos[0]) ** 2, axis=1)

    def body(i, state):
        idx, mind = state
        nxt = jnp.argmax(mind).astype(jnp.int32)
        idx = idx.at[i].set(nxt)
        d = jnp.sum((pos - pos[nxt]) ** 2, axis=1)
        mind = jnp.minimum(mind, d)
        return (idx, mind)

    idx, _ = jax.lax.fori_loop(1, n_samples, body, (idx0, d0))
    return idx


def _mlp_block(g_ref, c_ref, w_ref, b_ref, o_ref):
    g = g_ref[...]
    c = c_ref[...]
    h = jnp.maximum(g + c[:, None, :], 0.0)
    h2 = jnp.reshape(h, (QB * MAX_NBRS, 128)) @ w_ref[...]
    h2 = h2.reshape(QB, MAX_NBRS, 128)
    o_ref[...] = jnp.max(h2, axis=1) + b_ref[...]


def kernel(x, pos, batch, W1, b1, W2, b2):
    n_samples = int(N * RATIO)
    idx = _fps(pos, n_samples)
    pos_q = pos[idx]
    batch_q = batch[idx]

    d = (jnp.sum(pos_q ** 2, axis=1)[:, None]
         + jnp.sum(pos ** 2, axis=1)[None, :]
         - 2.0 * (pos_q @ pos.T))
    neg_vals, nbr = jax.lax.top_k(-d, MAX_NBRS)
    mask = (-neg_vals) <= (R * R)
    nbr = jnp.where(mask, nbr, idx[:, None])

    G = x @ W1[:D_FEAT] + pos @ W1[D_FEAT:] + b1
    c = -(pos_q @ W1[D_FEAT:])

    S = n_samples
    SP = ((S + QB - 1) // QB) * QB
    nbr_p = jnp.concatenate([nbr, jnp.zeros((SP - S, MAX_NBRS), jnp.int32)], 0)
    c_p = jnp.concatenate([c, jnp.zeros((SP - S, 128), jnp.float32)], 0)
    Gj = jnp.take(G, nbr_p, axis=0)

    out = pl.pallas_call(
        _mlp_block,
        grid=(SP // QB,),
        in_specs=[
            pl.BlockSpec((QB, MAX_NBRS, 128), lambda i: (i, 0, 0)),
            pl.BlockSpec((QB, 128), lambda i: (i, 0)),
            pl.BlockSpec((128, 128), lambda i: (0, 0)),
            pl.BlockSpec((1, 128), lambda i: (0, 0)),
        ],
        out_specs=pl.BlockSpec((QB, 128), lambda i: (i, 0)),
        out_shape=jax.ShapeDtypeStruct((SP, 128), jnp.float32),
    )(Gj, c_p, W2, b2.reshape(1, 128))

    return (out[:S], pos_q, batch_q)


# speedup vs baseline: 2.0787x; 2.0721x over previous
import functools

import jax
import jax.numpy as jnp
from jax.experimental import pallas as pl

RATIO = 0.25
R = 0.2
MAX_NBRS = 128
N = 10000
D_FEAT = 128

QB = 32


NPAD = 10240
FPS_ROWS = 8
FPS_COLS = NPAD // FPS_ROWS
S_SAMPLES = int(N * RATIO)
SPAD = 2560
SROWS = SPAD // 128


def _fps_body(px_ref, py_ref, pz_ref, out_ref):
    px = px_ref[...]
    py = py_ref[...]
    pz = pz_ref[...]
    shp = px.shape
    idx2d = (jax.lax.broadcasted_iota(jnp.int32, shp, 0) * FPS_COLS
             + jax.lax.broadcasted_iota(jnp.int32, shp, 1))
    valid = idx2d < N
    iota_s = (jax.lax.broadcasted_iota(jnp.int32, (SROWS, 128), 0) * 128
              + jax.lax.broadcasted_iota(jnp.int32, (SROWS, 128), 1))

    def dist_from(j):
        eq = idx2d == j
        vx = jnp.sum(jnp.where(eq, px, 0.0))
        vy = jnp.sum(jnp.where(eq, py, 0.0))
        vz = jnp.sum(jnp.where(eq, pz, 0.0))
        dx = px - vx
        dy = py - vy
        dz = pz - vz
        return (dx * dx + dz * dz) + dy * dy

    mind0 = jnp.where(valid, dist_from(0), -jnp.inf)
    carry0 = jnp.zeros((SROWS, 128), jnp.int32)

    def body(i, st):
        mind, out = st
        m = jnp.max(mind)
        cand = jnp.where(mind == m, idx2d, jnp.int32(2 ** 30))
        nxt = jnp.min(cand)
        mind = jnp.minimum(mind, dist_from(nxt))
        out = jnp.where(iota_s == i, nxt, out)
        return (mind, out)

    _, out = jax.lax.fori_loop(1, S_SAMPLES, body, (mind0, carry0))
    out_ref[...] = out


def _fps(pos, n_samples):
    pad = NPAD - N
    px = jnp.pad(pos[:, 0], (0, pad)).reshape(FPS_ROWS, FPS_COLS)
    py = jnp.pad(pos[:, 1], (0, pad)).reshape(FPS_ROWS, FPS_COLS)
    pz = jnp.pad(pos[:, 2], (0, pad)).reshape(FPS_ROWS, FPS_COLS)
    out = pl.pallas_call(
        _fps_body,
        out_shape=jax.ShapeDtypeStruct((SROWS, 128), jnp.int32),
    )(px, py, pz)
    return out.reshape(SPAD)[:n_samples]


def _mlp_block(g_ref, c_ref, w_ref, b_ref, o_ref):
    g = g_ref[...]
    c = c_ref[...]
    h = jnp.maximum(g + c[:, None, :], 0.0)
    h2 = jnp.reshape(h, (QB * MAX_NBRS, 128)) @ w_ref[...]
    h2 = h2.reshape(QB, MAX_NBRS, 128)
    o_ref[...] = jnp.max(h2, axis=1) + b_ref[...]


def kernel(x, pos, batch, W1, b1, W2, b2):
    n_samples = int(N * RATIO)
    idx = _fps(pos, n_samples)
    pos_q = pos[idx]
    batch_q = batch[idx]

    d = (jnp.sum(pos_q ** 2, axis=1)[:, None]
         + jnp.sum(pos ** 2, axis=1)[None, :]
         - 2.0 * (pos_q @ pos.T))
    neg_vals, nbr = jax.lax.top_k(-d, MAX_NBRS)
    mask = (-neg_vals) <= (R * R)
    nbr = jnp.where(mask, nbr, idx[:, None])

    G = x @ W1[:D_FEAT] + pos @ W1[D_FEAT:] + b1
    c = -(pos_q @ W1[D_FEAT:])

    S = n_samples
    SP = ((S + QB - 1) // QB) * QB
    nbr_p = jnp.concatenate([nbr, jnp.zeros((SP - S, MAX_NBRS), jnp.int32)], 0)
    c_p = jnp.concatenate([c, jnp.zeros((SP - S, 128), jnp.float32)], 0)
    Gj = jnp.take(G, nbr_p, axis=0)

    out = pl.pallas_call(
        _mlp_block,
        grid=(SP // QB,),
        in_specs=[
            pl.BlockSpec((QB, MAX_NBRS, 128), lambda i: (i, 0, 0)),
            pl.BlockSpec((QB, 128), lambda i: (i, 0)),
            pl.BlockSpec((128, 128), lambda i: (0, 0)),
            pl.BlockSpec((1, 128), lambda i: (0, 0)),
        ],
        out_specs=pl.BlockSpec((QB, 128), lambda i: (i, 0)),
        out_shape=jax.ShapeDtypeStruct((SP, 128), jnp.float32),
    )(Gj, c_p, W2, b2.reshape(1, 128))

    return (out[:S], pos_q, batch_q)
